# baseline (speedup 1.0000x reference)
import functools

import jax
import jax.numpy as jnp
from jax import lax
from jax.experimental import pallas as pl
from jax.experimental.pallas import tpu as pltpu
from jax.experimental.pallas import tpu_sc as plsc

B = 16384
D = 64
H = 128
NC = 2
NS = 16
NW = NC * NS
BPW = B // NW
CHUNK = 128
NCH = BPW // CHUNK

BLK = 2048


@functools.lru_cache(maxsize=None)
def _gather_fn():
    mesh = plsc.VectorSubcoreMesh(core_axis_name="c", subcore_axis_name="s")

    @functools.partial(
        pl.kernel,
        mesh=mesh,
        compiler_params=pltpu.CompilerParams(use_tc_tiling_on_sc=False),
        out_type=[
            jax.ShapeDtypeStruct((B, 2 * D), jnp.float32),
        ],
        scratch_types=[
            pltpu.VMEM((BPW,), jnp.int32),
            pltpu.VMEM((BPW,), jnp.int32),
            pltpu.VMEM((BPW, D), jnp.float32),
            pltpu.VMEM((BPW, D), jnp.float32),
            pltpu.SemaphoreType.DMA,
            pltpu.SemaphoreType.DMA,
        ],
    )
    def gather(user_hbm, book_hbm, utab_hbm, btab_hbm, ub_out,
               uidx, bidx, urows, brows, sem_u, sem_b):
        wid = lax.axis_index("s") * NC + lax.axis_index("c")
        base = wid * BPW
        pltpu.sync_copy(user_hbm.at[pl.ds(base, BPW)], uidx)
        pltpu.sync_copy(book_hbm.at[pl.ds(base, BPW)], bidx)
        copies = []
        for j in range(NCH):
            sl = pl.ds(j * CHUNK, CHUNK)
            copies.append(pltpu.async_copy(
                utab_hbm.at[uidx.at[sl]], urows.at[sl], sem_u))
            copies.append(pltpu.async_copy(
                btab_hbm.at[bidx.at[sl]], brows.at[sl], sem_b))
        for c in copies:
            c.wait()
        pltpu.sync_copy(urows, ub_out.at[pl.ds(base, BPW), pl.ds(0, D)])
        pltpu.sync_copy(brows, ub_out.at[pl.ds(base, BPW), pl.ds(D, D)])

    return gather


def _combine_body(tag_ref, w_ref, b_ref, ub_ref, out_ref):
    proj = jnp.dot(tag_ref[...], w_ref[...],
                   preferred_element_type=jnp.float32)
    integ = ub_ref[:, D:] + proj + b_ref[...]
    out_ref[...] = jnp.sum(ub_ref[:, :D] * integ, axis=1)


def _combine(tag, W, b2, UB):
    return pl.pallas_call(
        _combine_body,
        grid=(B // BLK,),
        in_specs=[
            pl.BlockSpec((BLK, H), lambda i: (i, 0)),
            pl.BlockSpec((H, D), lambda i: (0, 0)),
            pl.BlockSpec((1, D), lambda i: (0, 0)),
            pl.BlockSpec((BLK, 2 * D), lambda i: (i, 0)),
        ],
        out_specs=pl.BlockSpec((BLK,), lambda i: (i,)),
        out_shape=jax.ShapeDtypeStruct((B,), jnp.float32),
    )(tag, W, b2, UB)


def kernel(user, book, tag_embedding, user_table, book_table, W_lin, b_lin):
    UB, = _gather_fn()(user, book, user_table, book_table)
    return _combine(tag_embedding, W_lin, b_lin.reshape(1, D), UB)


# speedup vs baseline: 2.3062x; 2.3062x over previous
import functools

import jax
import jax.numpy as jnp
from jax import lax
from jax.experimental import pallas as pl
from jax.experimental.pallas import tpu as pltpu
from jax.experimental.pallas import tpu_sc as plsc

B = 16384
D = 64
H = 128
NC = 2
NS = 16
NW = NC * NS
BPW = B // NW
HPW = BPW // 2
NG = HPW // 16

BLK = 2048


@functools.lru_cache(maxsize=None)
def _gather_fn():
    mesh = plsc.VectorSubcoreMesh(core_axis_name="c", subcore_axis_name="s")

    @functools.partial(
        pl.kernel,
        mesh=mesh,
        out_type=[
            jax.ShapeDtypeStruct((B // 8, 8, D), jnp.float32),
            jax.ShapeDtypeStruct((B // 8, 8, D), jnp.float32),
        ],
        scratch_types=[
            pltpu.VMEM((BPW,), jnp.int32),
            pltpu.VMEM((BPW,), jnp.int32),
            pltpu.VMEM((BPW // 8, 8, D), jnp.float32),
            pltpu.SemaphoreType.DMA,
            pltpu.SemaphoreType.DMA,
        ],
    )
    def gather(user_hbm, book_hbm, utab3, btab3, u_out, b_out,
               uidx, bidx, buf, sem_u, sem_b):
        wid = lax.axis_index("s") * NC + lax.axis_index("c")
        base = wid * BPW
        pltpu.sync_copy(user_hbm.at[pl.ds(base, BPW)], uidx)
        pltpu.sync_copy(book_hbm.at[pl.ds(base, BPW)], bidx)

        def drain16(tab3, sem):
            for _ in range(16):
                pltpu.make_async_copy(
                    tab3.at[0, 0], buf.at[0, 0], sem).wait()

        def fire16(idxv, tab3, tile0, g, sem):
            tv = lax.shift_right_logical(idxv, 3)
            sv = lax.bitwise_and(idxv, 7)
            for lane in range(16):
                row = tile0 + g * 2 + lane // 8
                pltpu.async_copy(
                    tab3.at[tv[lane], sv[lane]],
                    buf.at[row, lane % 8], sem)

        for half in range(2):
            off = half * HPW

            def body(g, _):
                uv = uidx[pl.ds(off + g * 16, 16)]
                bv = bidx[pl.ds(off + g * 16, 16)]
                fire16(uv, utab3, 0, g, sem_u)
                fire16(bv, btab3, 32, g, sem_b)

                @pl.when(g > 0)
                def _d():
                    drain16(utab3, sem_u)
                    drain16(btab3, sem_b)
                return _
            lax.fori_loop(0, NG, body, None)
            drain16(utab3, sem_u)
            drain16(btab3, sem_b)
            pltpu.sync_copy(
                buf.at[pl.ds(0, HPW // 8)],
                u_out.at[pl.ds((base + off) // 8, HPW // 8)])
            pltpu.sync_copy(
                buf.at[pl.ds(32, HPW // 8)],
                b_out.at[pl.ds((base + off) // 8, HPW // 8)])

    return gather


def _combine_body(tag_ref, w_ref, b_ref, u_ref, bk_ref, out_ref):
    proj = jnp.dot(tag_ref[...], w_ref[...],
                   preferred_element_type=jnp.float32)
    integ = bk_ref[...] + proj + b_ref[...]
    out_ref[...] = jnp.sum(u_ref[...] * integ, axis=1)


def _combine(tag, W, b2, U, Bk):
    return pl.pallas_call(
        _combine_body,
        grid=(B // BLK,),
        in_specs=[
            pl.BlockSpec((BLK, H), lambda i: (i, 0)),
            pl.BlockSpec((H, D), lambda i: (0, 0)),
            pl.BlockSpec((1, D), lambda i: (0, 0)),
            pl.BlockSpec((BLK, D), lambda i: (i, 0)),
            pl.BlockSpec((BLK, D), lambda i: (i, 0)),
        ],
        out_specs=pl.BlockSpec((BLK,), lambda i: (i,)),
        out_shape=jax.ShapeDtypeStruct((B,), jnp.float32),
    )(tag, W, b2, U, Bk)


def kernel(user, book, tag_embedding, user_table, book_table, W_lin, b_lin):
    U3, Bk3 = _gather_fn()(
        user, book,
        user_table.reshape(user_table.shape[0] // 8, 8, D),
        book_table.reshape(book_table.shape[0] // 8, 8, D))
    return _combine(tag_embedding, W_lin, b_lin.reshape(1, D),
                    U3.reshape(B, D), Bk3.reshape(B, D))


# speedup vs baseline: 2.3152x; 1.0039x over previous
import functools

import jax
import jax.numpy as jnp
from jax import lax
from jax.experimental import pallas as pl
from jax.experimental.pallas import tpu as pltpu
from jax.experimental.pallas import tpu_sc as plsc

B = 16384
D = 64
H = 128
NC = 2
NS = 16
NW = NC * NS
BPW = B // NW
HPW = BPW // 2
NG = HPW // 16

BLK = 2048


def _proj_body(tag_ref, w_ref, b_ref, out_ref):
    p = jnp.dot(tag_ref[...], w_ref[...],
                preferred_element_type=jnp.float32) + b_ref[...]
    out_ref[...] = jnp.concatenate([p, p], axis=1)


def _proj(tag, W, b2):
    return pl.pallas_call(
        _proj_body,
        grid=(B // BLK,),
        in_specs=[
            pl.BlockSpec((BLK, H), lambda i: (i, 0)),
            pl.BlockSpec((H, D), lambda i: (0, 0)),
            pl.BlockSpec((1, D), lambda i: (0, 0)),
        ],
        out_specs=pl.BlockSpec((BLK, 2 * D), lambda i: (i, 0)),
        out_shape=jax.ShapeDtypeStruct((B, 2 * D), jnp.float32),
    )(tag, W, b2)


@functools.lru_cache(maxsize=None)
def _gather_fn():
    mesh = plsc.VectorSubcoreMesh(core_axis_name="c", subcore_axis_name="s")

    @functools.partial(
        pl.kernel,
        mesh=mesh,
        compiler_params=pltpu.CompilerParams(needs_layout_passes=False),
        out_type=jax.ShapeDtypeStruct((B,), jnp.float32),
        scratch_types=[
            pltpu.VMEM((BPW,), jnp.int32),
            pltpu.VMEM((BPW,), jnp.int32),
            pltpu.VMEM((BPW // 8, 8, D), jnp.float32),
            pltpu.VMEM((HPW, 2 * D), jnp.float32),
            pltpu.VMEM((BPW,), jnp.float32),
            pltpu.SemaphoreType.DMA,
            pltpu.SemaphoreType.DMA,
            pltpu.SemaphoreType.DMA,
        ],
    )
    def gather(user_hbm, book_hbm, utab3, btab3, proj_hbm, out_hbm,
               uidx, bidx, buf, pstage, res, sem_u, sem_b, sem_p):
        wid = lax.axis_index("s") * NC + lax.axis_index("c")
        base = wid * BPW
        pltpu.sync_copy(user_hbm.at[pl.ds(base, BPW)], uidx)
        pltpu.sync_copy(book_hbm.at[pl.ds(base, BPW)], bidx)

        def drain16(tab3, sem):
            for _ in range(16):
                pltpu.make_async_copy(
                    tab3.at[0, 0], buf.at[0, 0], sem).wait()

        def fire16(idxv, tab3, tile0, g, sem):
            tv = lax.shift_right_logical(idxv, 3)
            sv = lax.bitwise_and(idxv, 7)
            for lane in range(16):
                row = tile0 + g * 2 + lane // 8
                pltpu.async_copy(
                    tab3.at[tv[lane], sv[lane]],
                    buf.at[row, lane % 8], sem)

        lane_iota = lax.iota(jnp.int32, 16)

        for half in range(2):
            off = half * HPW
            pcopy = pltpu.async_copy(
                proj_hbm.at[pl.ds(base + off, HPW)], pstage, sem_p)

            def body(g, _):
                uv = uidx[pl.ds(off + g * 16, 16)]
                bv = bidx[pl.ds(off + g * 16, 16)]
                fire16(uv, utab3, 0, g, sem_u)
                fire16(bv, btab3, 32, g, sem_b)

                @pl.when(g > 0)
                def _d():
                    drain16(utab3, sem_u)
                    drain16(btab3, sem_b)
                return _
            lax.fori_loop(0, NG, body, None)
            drain16(utab3, sem_u)
            drain16(btab3, sem_b)
            pcopy.wait()

            def dot16(c, _):
                outv = jnp.zeros((16,), jnp.float32)
                for lane in range(16):
                    tile = c * 2 + lane // 8
                    s = lane % 8
                    acc = jnp.zeros((16,), jnp.float32)
                    for k in range(D // 16):
                        sl = pl.ds(k * 16, 16)
                        u = buf[tile, s, sl]
                        bk = buf[32 + tile, s, sl]
                        p = pstage[c * 16 + lane, sl]
                        acc = acc + u * (bk + p)
                    tot = jnp.sum(acc)
                    outv = jnp.where(lane_iota == lane, tot, outv)
                res[pl.ds(off + c * 16, 16)] = outv
                return _
            lax.fori_loop(0, NG, dot16, None)

        pltpu.sync_copy(res, out_hbm.at[pl.ds(base, BPW)])

    return gather


def kernel(user, book, tag_embedding, user_table, book_table, W_lin, b_lin):
    proj = _proj(tag_embedding, W_lin, b_lin.reshape(1, D))
    return _gather_fn()(
        user, book,
        user_table.reshape(user_table.shape[0] // 8, 8, D),
        book_table.reshape(book_table.shape[0] // 8, 8, D),
        proj)
